# baseline (speedup 1.0000x reference)
import functools

import jax
import jax.numpy as jnp
from jax import lax
from jax.experimental import pallas as pl
from jax.experimental.pallas import tpu as pltpu
from jax.experimental.pallas import tpu_sc as plsc

_B, _H, _GY, _GX = 32, 3, 160, 160
_NTOT = _B * _H * _GY * _GX
_NA = 20000
_NC, _NS = 2, 16
_NW = _NC * _NS
_OWN = _NTOT // _NW
_ROWS = _H * _GY
_SROWS = 48
_NCHUNK = _ROWS // _SROWS
_GRP = _NA // 16
_U = 10
_CPR = _GX // 16

_LOG1P_C = (9.08378684e-08, 9.99991455e-01, -4.99801163e-01, 3.31334006e-01,
            -2.39190717e-01, 1.64783497e-01, -9.23137687e-02, 3.44185935e-02,
            -6.07487764e-03)

_mesh = plsc.VectorSubcoreMesh(core_axis_name="c", subcore_axis_name="s")


@functools.partial(
    pl.kernel,
    mesh=_mesh,
    compiler_params=pltpu.CompilerParams(needs_layout_passes=False),
    out_type=[
        jax.ShapeDtypeStruct((_NW * 16,), jnp.float32),
        jax.ShapeDtypeStruct((_NW * 16,), jnp.float32),
    ],
    scratch_types=[
        pltpu.VMEM((_NA,), jnp.int32),
        pltpu.VMEM((_OWN,), jnp.int32),
        pltpu.VMEM((_SROWS, _GX), jnp.float32),
        pltpu.VMEM((_SROWS, _GX), jnp.float32),
        pltpu.VMEM((16,), jnp.float32),
        pltpu.VMEM((16,), jnp.float32),
        pltpu.SemaphoreType.DMA,
        pltpu.SemaphoreType.DMA,
        pltpu.SemaphoreType.DMA,
    ],
)
def _sc_loss_parts(x_hbm, idx_hbm, sp_out, mk_out, idx_v, table_v,
                   buf_a, buf_b, sp_v, mk_v, sem_i, sem_a, sem_b):
    wid = lax.axis_index("s") * _NC + lax.axis_index("c")
    base = wid * _OWN
    idx_dma = pltpu.async_copy(idx_hbm, idx_v, sem_i)

    zero16 = jnp.zeros((16,), jnp.int32)
    one16 = jnp.full((16,), 0x3F800000, jnp.int32)

    def _init(i, c):
        for k in range(2 * _U):
            g = i * 2 * _U + k
            table_v[pl.ds(pl.multiple_of(g * 16, 16), 16)] = zero16
        return c

    lax.fori_loop(0, _OWN // 16 // (2 * _U), _init, jnp.int32(0))
    idx_dma.wait()

    bufs = [buf_a, buf_b]
    sems = [sem_a, sem_b]

    def _chunk_dma(c):
        return pltpu.async_copy(
            x_hbm.at[wid, pl.ds(c * _SROWS, _SROWS)], bufs[c % 2], sems[c % 2]
        )

    dmas = {0: _chunk_dma(0), 1: _chunk_dma(1)}

    def _mark(i, c):
        for k in range(_U):
            g = i * _U + k
            idx16 = idx_v[pl.ds(pl.multiple_of(g * 16, 16), 16)]
            rel = idx16 - base
            m = (rel >= 0) & (rel < _OWN)
            relc = jnp.where(m, rel, 0)
            plsc.store_scatter(table_v, [relc], one16, mask=m)
        return c

    lax.fori_loop(0, _GRP // _U, _mark, jnp.int32(0))

    sp_acc = jnp.zeros((16,), jnp.float32)
    mk_acc = jnp.zeros((16,), jnp.float32)
    for c in range(_NCHUNK):
        dmas[c].wait()
        if c + 2 < _NCHUNK:
            dmas[c + 2] = _chunk_dma(c + 2)
        buf = bufs[c % 2]

        def _dense(i, carry, c=c, buf=buf):
            sp, mk = carry
            for rr in range(2):
                r = i * 2 + rr
                trow = c * _SROWS * _GX + r * _GX
                for k in range(_CPR):
                    xg = buf[r, pl.ds(k * 16, 16)]
                    tb = table_v[pl.ds(pl.multiple_of(trow + k * 16, 16), 16)]
                    e = jnp.exp(-jnp.abs(xg))
                    gpoly = jnp.float32(_LOG1P_C[8])
                    for cc in _LOG1P_C[7::-1]:
                        gpoly = gpoly * e + jnp.float32(cc)
                    sp = sp + (jnp.maximum(xg, 0.0) + gpoly)
                    mk = mk + xg * plsc.bitcast(tb, jnp.float32)
            return sp, mk

        sp_acc, mk_acc = lax.fori_loop(0, _SROWS // 2, _dense,
                                       (sp_acc, mk_acc))

    sp_v[...] = sp_acc
    mk_v[...] = mk_acc
    pltpu.sync_copy(sp_v, sp_out.at[pl.ds(wid * 16, 16)])
    pltpu.sync_copy(mk_v, mk_out.at[pl.ds(wid * 16, 16)])


def _combine_body(sp_ref, mk_ref, out_ref):
    out_ref[0, 0] = (jnp.sum(sp_ref[...]) - jnp.sum(mk_ref[...])) / _NTOT


_combine = pl.pallas_call(
    _combine_body,
    in_specs=[
        pl.BlockSpec((4, 128), lambda: (0, 0)),
        pl.BlockSpec((4, 128), lambda: (0, 0)),
    ],
    out_specs=pl.BlockSpec((1, 1), lambda: (0, 0), memory_space=pltpu.SMEM),
    out_shape=jax.ShapeDtypeStruct((1, 1), jnp.float32),
)


def kernel(pre_activation_o, img_idxs, head_idxs, grid_y_idxs, grid_x_idxs):
    flat = (
        (img_idxs.astype(jnp.int32) * _H + head_idxs) * _GY + grid_y_idxs
    ) * _GX + grid_x_idxs
    x3d = pre_activation_o.reshape(_B, _ROWS, _GX)
    sp, mk = _sc_loss_parts(x3d, flat)
    return _combine(sp.reshape(4, 128), mk.reshape(4, 128))[0, 0]


# speedup vs baseline: 1.1827x; 1.1827x over previous
import functools

import jax
import jax.numpy as jnp
from jax import lax
from jax.experimental import pallas as pl
from jax.experimental.pallas import tpu as pltpu
from jax.experimental.pallas import tpu_sc as plsc

_B, _H, _GY, _GX = 32, 3, 160, 160
_NTOT = _B * _H * _GY * _GX
_NA = 20000
_NC, _NS = 2, 16
_NW = _NC * _NS
_OWN = _NTOT // _NW
_ROWS = _H * _GY
_SROWS = 48
_NCHUNK = _ROWS // _SROWS
_GRP = _NA // 16
_U = 10
_CPR = _GX // 16

_LOG1P_C = (9.08378684e-08, 9.99991455e-01, -4.99801163e-01, 3.31334006e-01,
            -2.39190717e-01, 1.64783497e-01, -9.23137687e-02, 3.44185935e-02,
            -6.07487764e-03)

_mesh = plsc.VectorSubcoreMesh(core_axis_name="c", subcore_axis_name="s")


@functools.partial(
    pl.kernel,
    mesh=_mesh,
    compiler_params=pltpu.CompilerParams(needs_layout_passes=False),
    out_type=[
        jax.ShapeDtypeStruct((_NW * 16,), jnp.float32),
        jax.ShapeDtypeStruct((_NW * 16,), jnp.float32),
    ],
    scratch_types=[
        pltpu.VMEM((_NA,), jnp.int32),
        pltpu.VMEM((_OWN,), jnp.int32),
        pltpu.VMEM((_SROWS, _GX), jnp.float32),
        pltpu.VMEM((_SROWS, _GX), jnp.float32),
        pltpu.VMEM((16,), jnp.float32),
        pltpu.VMEM((16,), jnp.float32),
        pltpu.SemaphoreType.DMA,
        pltpu.SemaphoreType.DMA,
        pltpu.SemaphoreType.DMA,
    ],
)
def _sc_loss_parts(x_hbm, idx_hbm, sp_out, mk_out, idx_v, table_v,
                   buf_a, buf_b, sp_v, mk_v, sem_i, sem_a, sem_b):
    wid = lax.axis_index("s") * _NC + lax.axis_index("c")
    base = wid * _OWN
    idx_dma = pltpu.async_copy(idx_hbm, idx_v, sem_i)

    zero16 = jnp.zeros((16,), jnp.int32)
    one16 = jnp.full((16,), 0x3F800000, jnp.int32)

    def _init(i, c):
        for k in range(2 * _U):
            g = i * 2 * _U + k
            table_v[pl.ds(pl.multiple_of(g * 16, 16), 16)] = zero16
        return c

    lax.fori_loop(0, _OWN // 16 // (2 * _U), _init, jnp.int32(0))
    idx_dma.wait()

    bufs = [buf_a, buf_b]
    sems = [sem_a, sem_b]

    def _chunk_dma(c):
        return pltpu.async_copy(
            x_hbm.at[wid, pl.ds(c * _SROWS, _SROWS)], bufs[c % 2], sems[c % 2]
        )

    dmas = {0: _chunk_dma(0), 1: _chunk_dma(1)}

    def _mark(i, c):
        for k in range(_U):
            g = i * _U + k
            idx16 = idx_v[pl.ds(pl.multiple_of(g * 16, 16), 16)]
            rel = idx16 - base
            m = (rel >= 0) & (rel < _OWN)
            relc = jnp.where(m, rel, 0)
            plsc.store_scatter(table_v, [relc], one16, mask=m)
        return c

    lax.fori_loop(0, _GRP // _U, _mark, jnp.int32(0))

    sp_acc = jnp.zeros((16,), jnp.float32)
    mk_acc = jnp.zeros((16,), jnp.float32)
    for c in range(_NCHUNK):
        dmas[c].wait()
        if c + 2 < _NCHUNK:
            dmas[c + 2] = _chunk_dma(c + 2)
        buf = bufs[c % 2]

        def _dense(r, carry, c=c, buf=buf):
            sp, mk = carry
            trow = c * _SROWS * _GX + r * _GX
            for k in range(_CPR):
                xg = buf[r, pl.ds(k * 16, 16)]
                tb = table_v[pl.ds(pl.multiple_of(trow + k * 16, 16), 16)]
                e = jnp.exp(-jnp.abs(xg))
                gpoly = jnp.float32(_LOG1P_C[8])
                for cc in _LOG1P_C[7::-1]:
                    gpoly = gpoly * e + jnp.float32(cc)
                sp = sp + (jnp.maximum(xg, 0.0) + gpoly)
                mk = mk + xg * plsc.bitcast(tb, jnp.float32)
            return sp, mk

        sp_acc, mk_acc = lax.fori_loop(0, _SROWS, _dense, (sp_acc, mk_acc))

    sp_v[...] = sp_acc
    mk_v[...] = mk_acc
    pltpu.sync_copy(sp_v, sp_out.at[pl.ds(wid * 16, 16)])
    pltpu.sync_copy(mk_v, mk_out.at[pl.ds(wid * 16, 16)])


def _combine_body(sp_ref, mk_ref, out_ref):
    out_ref[0, 0] = (jnp.sum(sp_ref[...]) - jnp.sum(mk_ref[...])) / _NTOT


_combine = pl.pallas_call(
    _combine_body,
    in_specs=[
        pl.BlockSpec((4, 128), lambda: (0, 0)),
        pl.BlockSpec((4, 128), lambda: (0, 0)),
    ],
    out_specs=pl.BlockSpec((1, 1), lambda: (0, 0), memory_space=pltpu.SMEM),
    out_shape=jax.ShapeDtypeStruct((1, 1), jnp.float32),
)


def kernel(pre_activation_o, img_idxs, head_idxs, grid_y_idxs, grid_x_idxs):
    flat = (
        (img_idxs.astype(jnp.int32) * _H + head_idxs) * _GY + grid_y_idxs
    ) * _GX + grid_x_idxs
    x3d = pre_activation_o.reshape(_B, _ROWS, _GX)
    sp, mk = _sc_loss_parts(x3d, flat)
    return _combine(sp.reshape(4, 128), mk.reshape(4, 128))[0, 0]
